# baseline (speedup 1.0000x reference)
import functools

import jax
import jax.numpy as jnp
from jax import lax
from jax.experimental import pallas as pl
from jax.experimental.pallas import tpu as pltpu
from jax.experimental.pallas import tpu_sc as plsc

N = 10000
E = 320000
D = 128

NC = 2
NS = 16
NW = NC * NS
EPW = E // NW
K = 128
CH = 79
CHP = 80
EPWP = CHP * K
RPT = 624
MB = 1000


def _mm_body(x_ref, w_ref, o_ref):
    o_ref[...] = jnp.dot(x_ref[...], w_ref[...],
                         preferred_element_type=jnp.float32)


def _combine_body(a_ref, b_ref, o_ref):
    o_ref[...] = jnp.maximum(a_ref[0] + b_ref[0], 0.0)


_mesh = plsc.VectorSubcoreMesh(core_axis_name="c", subcore_axis_name="s")


@functools.partial(
    pl.kernel,
    mesh=_mesh,
    out_type=jax.ShapeDtypeStruct((NC, N, D), jnp.float32),
    scratch_types=[
        pltpu.VMEM((4, K), jnp.int32),
        pltpu.VMEM((K,), jnp.int32),
        pltpu.VMEM((K,), jnp.int32),
        pltpu.VMEM((K,), jnp.int32),
        pltpu.VMEM((K,), jnp.int32),
        pltpu.VMEM((K,), jnp.int32),
        pltpu.VMEM((K,), jnp.int32),
        pltpu.VMEM((2, K), jnp.float32),
        pltpu.VMEM((K, D), jnp.float32),
        pltpu.VMEM((K, D), jnp.float32),
        pltpu.VMEM((K, D), jnp.float32),
        pltpu.VMEM_SHARED((N, D), jnp.float32),
        pltpu.SemaphoreType.DMA,
        pltpu.SemaphoreType.DMA,
        pltpu.SemaphoreType.DMA,
        pltpu.SemaphoreType.DMA,
    ],
)
def _sc_scatter(presup, src, dst, w, zeros, zeros16, out,
                src_v, d0, d1, d2, d3, d4, d5, w_v, r0, r1, r2, acc,
                sem_g, sem_i, sem_w, sem_sc):
    cid = lax.axis_index("c")
    sid = lax.axis_index("s")
    wid = cid * NS + sid
    dst_bufs = (d0, d1, d2, d3, d4, d5)
    row_bufs = (r0, r1, r2)

    pltpu.async_copy(zeros, acc.at[pl.ds(sid * RPT, RPT)], sem_sc)
    pltpu.make_async_copy(zeros, acc.at[pl.ds(sid * RPT, RPT)], sem_sc).wait()

    @pl.when(sid == NS - 1)
    def _():
        pltpu.async_copy(zeros16, acc.at[pl.ds(NS * RPT, 16)], sem_sc)
        pltpu.make_async_copy(zeros16, acc.at[pl.ds(NS * RPT, 16)],
                              sem_sc).wait()
    plsc.subcore_barrier()

    def issue_idx(j, u):
        pltpu.async_copy(src.at[wid, j], src_v.at[u % 3], sem_i)
        pltpu.async_copy(dst.at[wid, j], dst_bufs[u % 6], sem_i)

    def wait_idx(u):
        pltpu.make_async_copy(src.at[0, 0], src_v.at[u % 3], sem_i).wait()
        pltpu.make_async_copy(dst.at[0, 0], dst_bufs[u % 6], sem_i).wait()

    def issue_w(j, u):
        pltpu.async_copy(w.at[wid, j], w_v.at[u % 2], sem_w)

    def wait_w(u):
        pltpu.make_async_copy(w.at[0, 0], w_v.at[u % 2], sem_w).wait()

    def issue_gather(u):
        pltpu.async_copy(presup.at[src_v.at[u % 3]], row_bufs[u % 3],
                         sem_g)

    def wait_gather(u):
        pltpu.make_async_copy(presup.at[pl.ds(0, K)], row_bufs[u % 3],
                              sem_g).wait()

    def scale(u):
        rv = row_bufs[u % 3]
        bw = u % 2

        @plsc.parallel_loop(0, K, step=16)
        def _scale(k0):
            grp = w_v[bw, pl.ds(k0, 16)]
            for i in range(16):
                wv = jnp.broadcast_to(grp[i], (16,))
                for c in range(D // 16):
                    sl = pl.ds(c * 16, 16)
                    rv[k0 + i, sl] = rv[k0 + i, sl] * wv

    def issue_scatter(u):
        pltpu.async_copy(row_bufs[u % 3], acc.at[dst_bufs[u % 6]],
                         sem_sc, add=True)

    def wait_scatter():
        pltpu.make_async_copy(row_bufs[0], acc.at[pl.ds(0, K)],
                              sem_sc).wait()

    issue_idx(0, 0)
    issue_idx(1, 1)
    issue_idx(2, 2)
    issue_w(0, 0)
    wait_idx(0)
    issue_gather(0)
    wait_idx(1)
    issue_gather(1)
    issue_w(1, 1)
    wait_w(0)
    wait_gather(0)
    scale(0)
    issue_scatter(0)
    issue_idx(3, 3)
    wait_idx(2)
    issue_gather(2)
    issue_w(2, 2)
    wait_w(1)
    wait_gather(1)
    scale(1)
    issue_scatter(1)
    issue_idx(4, 4)

    def chunk_body(j, u):
        wait_scatter()
        wait_idx(u + 1)
        issue_gather(u + 1)
        issue_w(j + 1, u + 1)
        wait_w(u)
        wait_gather(u)
        scale(u)
        issue_scatter(u)

    def hex_body(t, carry):
        for u in (2, 3, 4, 5, 6, 7):
            j = 6 * t + u
            chunk_body(j, u)
            issue_idx(j + 3, u + 3)
        return carry
    lax.fori_loop(0, (CH - 5) // 6, hex_body, 0)

    chunk_body(CH - 5, CH - 5)
    issue_idx(CH - 2, CH - 2)
    chunk_body(CH - 4, CH - 4)
    issue_idx(CH - 1, CH - 1)
    chunk_body(CH - 3, CH - 3)
    chunk_body(CH - 2, CH - 2)
    wait_scatter()
    wait_w(CH - 1)
    wait_gather(CH - 1)
    scale(CH - 1)
    issue_scatter(CH - 1)

    wait_scatter()
    wait_scatter()

    plsc.subcore_barrier()
    pltpu.sync_copy(acc.at[pl.ds(sid * RPT, RPT)],
                    out.at[cid, pl.ds(sid * RPT, RPT)])

    @pl.when(sid == NS - 1)
    def _():
        pltpu.sync_copy(acc.at[pl.ds(NS * RPT, 16)],
                        out.at[cid, pl.ds(NS * RPT, 16)])


def kernel(x, edge_index, edge_weight, W):
    pre_sup = pl.pallas_call(
        _mm_body,
        grid=(N // MB,),
        in_specs=[
            pl.BlockSpec((MB, D), lambda i: (i, 0)),
            pl.BlockSpec((D, D), lambda i: (0, 0)),
        ],
        out_specs=pl.BlockSpec((MB, D), lambda i: (i, 0)),
        out_shape=jax.ShapeDtypeStruct((N, D), jnp.float32),
    )(x, W)

    pad = ((0, 0), (0, EPWP - EPW))
    src = jnp.pad(edge_index[0].astype(jnp.int32).reshape(NW, EPW),
                  pad).reshape(NW, CHP, K)
    dst = jnp.pad(edge_index[1].astype(jnp.int32).reshape(NW, EPW),
                  pad).reshape(NW, CHP, K)
    zeros = jnp.zeros((RPT, D), jnp.float32)
    zeros16 = jnp.zeros((16, D), jnp.float32)
    wm = jnp.pad(edge_weight.astype(jnp.float32).reshape(NW, EPW),
                 pad).reshape(NW, CHP, K)

    partials = _sc_scatter(pre_sup, src, dst, wm, zeros, zeros16)

    out = pl.pallas_call(
        _combine_body,
        grid=(N // MB,),
        in_specs=[
            pl.BlockSpec((1, MB, D), lambda i: (0, i, 0)),
            pl.BlockSpec((1, MB, D), lambda i: (1, i, 0)),
        ],
        out_specs=pl.BlockSpec((MB, D), lambda i: (i, 0)),
        out_shape=jax.ShapeDtypeStruct((N, D), jnp.float32),
    )(partials, partials)
    return out


# speedup vs baseline: 1.5302x; 1.5302x over previous
import functools

import jax
import jax.numpy as jnp
from jax import lax
from jax.experimental import pallas as pl
from jax.experimental.pallas import tpu as pltpu
from jax.experimental.pallas import tpu_sc as plsc

N = 10000
E = 320000
D = 128

NC = 2
NS = 16
NW = NC * NS
EPW = E // NW
K = 80
CH = EPW // K
CHP = 128
RPT = 624
MB = 1000


def _mm_body(x_ref, w_ref, o_ref):
    o_ref[...] = jnp.dot(x_ref[...], w_ref[...],
                         preferred_element_type=jnp.float32)


def _combine_body(a_ref, b_ref, o_ref):
    o_ref[...] = jnp.maximum(a_ref[0] + b_ref[0], 0.0)


_mesh = plsc.VectorSubcoreMesh(core_axis_name="c", subcore_axis_name="s")


@functools.partial(
    pl.kernel,
    mesh=_mesh,
    out_type=jax.ShapeDtypeStruct((NC, N, D), jnp.float32),
    scratch_types=[
        pltpu.VMEM((4, K), jnp.int32),
        pltpu.VMEM((K,), jnp.int32),
        pltpu.VMEM((K,), jnp.int32),
        pltpu.VMEM((K,), jnp.int32),
        pltpu.VMEM((K,), jnp.int32),
        pltpu.VMEM((K,), jnp.int32),
        pltpu.VMEM((K,), jnp.int32),
        pltpu.VMEM((2, K), jnp.float32),
        pltpu.VMEM((K, D), jnp.float32),
        pltpu.VMEM((K, D), jnp.float32),
        pltpu.VMEM((K, D), jnp.float32),
        pltpu.VMEM_SHARED((N, D), jnp.float32),
        pltpu.SemaphoreType.DMA,
        pltpu.SemaphoreType.DMA,
        pltpu.SemaphoreType.DMA,
        pltpu.SemaphoreType.DMA,
    ],
)
def _sc_scatter(presup, src, dst, w, zeros, zeros16, out,
                src_v, d0, d1, d2, d3, d4, d5, w_v, r0, r1, r2, acc,
                sem_g, sem_i, sem_w, sem_sc):
    cid = lax.axis_index("c")
    sid = lax.axis_index("s")
    wid = cid * NS + sid
    dst_bufs = (d0, d1, d2, d3, d4, d5)
    row_bufs = (r0, r1, r2)

    pltpu.async_copy(zeros, acc.at[pl.ds(sid * RPT, RPT)], sem_sc)

    @pl.when(sid == NS - 1)
    def _():
        pltpu.async_copy(zeros16, acc.at[pl.ds(NS * RPT, 16)], sem_sc)

    def issue_idx(j, u):
        pltpu.async_copy(src.at[wid, j], src_v.at[u % 3], sem_i)
        pltpu.async_copy(dst.at[wid, j], dst_bufs[u % 6], sem_i)

    def wait_idx(u):
        pltpu.make_async_copy(src.at[0, 0], src_v.at[u % 3], sem_i).wait()
        pltpu.make_async_copy(dst.at[0, 0], dst_bufs[u % 6], sem_i).wait()

    def issue_w(j, u):
        pltpu.async_copy(w.at[wid, j], w_v.at[u % 2], sem_w)

    def wait_w(u):
        pltpu.make_async_copy(w.at[0, 0], w_v.at[u % 2], sem_w).wait()

    def issue_gather(u):
        pltpu.async_copy(presup.at[src_v.at[u % 3]], row_bufs[u % 3],
                         sem_g)

    def wait_gather(u):
        pltpu.make_async_copy(presup.at[pl.ds(0, K)], row_bufs[u % 3],
                              sem_g).wait()

    def scale(u):
        rv = row_bufs[u % 3]
        bw = u % 2

        @plsc.parallel_loop(0, K, step=16)
        def _scale(k0):
            grp = w_v[bw, pl.ds(k0, 16)]
            for i in range(16):
                wv = jnp.broadcast_to(grp[i], (16,))
                for c in range(D // 16):
                    sl = pl.ds(c * 16, 16)
                    rv[k0 + i, sl] = rv[k0 + i, sl] * wv

    def issue_scatter(u):
        pltpu.async_copy(row_bufs[u % 3], acc.at[dst_bufs[u % 6]],
                         sem_sc, add=True)

    def wait_scatter():
        pltpu.make_async_copy(row_bufs[0], acc.at[pl.ds(0, K)],
                              sem_sc).wait()

    issue_idx(0, 0)
    issue_idx(1, 1)
    issue_idx(2, 2)
    issue_w(0, 0)
    wait_idx(0)
    issue_gather(0)
    wait_idx(1)
    issue_gather(1)
    issue_w(1, 1)
    wait_w(0)
    wait_gather(0)
    scale(0)
    pltpu.make_async_copy(zeros, acc.at[pl.ds(sid * RPT, RPT)], sem_sc).wait()

    @pl.when(sid == NS - 1)
    def _():
        pltpu.make_async_copy(zeros16, acc.at[pl.ds(NS * RPT, 16)],
                              sem_sc).wait()
    plsc.subcore_barrier()
    issue_scatter(0)
    issue_idx(3, 3)
    wait_idx(2)
    issue_gather(2)
    issue_w(2, 2)
    wait_w(1)
    wait_gather(1)
    scale(1)
    issue_scatter(1)
    issue_idx(4, 4)

    def chunk_body(j, u):
        wait_scatter()
        wait_idx(u + 1)
        issue_gather(u + 1)
        issue_w(j + 1, u + 1)
        wait_w(u)
        wait_gather(u)
        scale(u)
        issue_scatter(u)

    def hex_body(t, carry):
        for u in (2, 3, 4, 5, 6, 7):
            j = 6 * t + u
            chunk_body(j, u)
            issue_idx(j + 3, u + 3)
        return carry
    lax.fori_loop(0, (CH - 5) // 6, hex_body, 0)

    chunk_body(CH - 3, CH - 3)
    chunk_body(CH - 2, CH - 2)
    wait_scatter()
    wait_w(CH - 1)
    wait_gather(CH - 1)
    scale(CH - 1)
    issue_scatter(CH - 1)

    wait_scatter()
    wait_scatter()

    plsc.subcore_barrier()
    pltpu.sync_copy(acc.at[pl.ds(sid * RPT, RPT)],
                    out.at[cid, pl.ds(sid * RPT, RPT)])

    @pl.when(sid == NS - 1)
    def _():
        pltpu.sync_copy(acc.at[pl.ds(NS * RPT, 16)],
                        out.at[cid, pl.ds(NS * RPT, 16)])


def kernel(x, edge_index, edge_weight, W):
    pre_sup = pl.pallas_call(
        _mm_body,
        grid=(N // MB,),
        in_specs=[
            pl.BlockSpec((MB, D), lambda i: (i, 0)),
            pl.BlockSpec((D, D), lambda i: (0, 0)),
        ],
        out_specs=pl.BlockSpec((MB, D), lambda i: (i, 0)),
        out_shape=jax.ShapeDtypeStruct((N, D), jnp.float32),
    )(x, W)

    pad = ((0, 0), (0, CHP - CH), (0, 0))
    src = jnp.pad(edge_index[0].astype(jnp.int32).reshape(NW, CH, K), pad)
    dst = jnp.pad(edge_index[1].astype(jnp.int32).reshape(NW, CH, K), pad)
    zeros = jnp.zeros((RPT, D), jnp.float32)
    zeros16 = jnp.zeros((16, D), jnp.float32)
    wm = jnp.pad(edge_weight.astype(jnp.float32).reshape(NW, CH, K), pad)

    partials = _sc_scatter(pre_sup, src, dst, wm, zeros, zeros16)

    out = pl.pallas_call(
        _combine_body,
        grid=(N // MB,),
        in_specs=[
            pl.BlockSpec((1, MB, D), lambda i: (0, i, 0)),
            pl.BlockSpec((1, MB, D), lambda i: (1, i, 0)),
        ],
        out_specs=pl.BlockSpec((MB, D), lambda i: (i, 0)),
        out_shape=jax.ShapeDtypeStruct((N, D), jnp.float32),
    )(partials, partials)
    return out
